# Initial kernel scaffold; baseline (speedup 1.0000x reference)
import jax
import jax.numpy as jnp
from jax.experimental import pallas as pl


def kernel(x, edge_index, W_gc, b_gc, W_fair, b_fair, W_cls, b_cls):
    raise NotImplementedError("write your pallas kernel here")


# speedup vs baseline: 7.4383x; 7.4383x over previous
import functools

import jax
import jax.numpy as jnp
from jax import lax
from jax.experimental import pallas as pl
from jax.experimental.pallas import tpu as pltpu
from jax.experimental.pallas import tpu_sc as plsc

_N = 10000
_E = 320000
_D = 128
_H = 128

_NC = 2
_NS = 16
_CH = 80
_NJ = _E // (_NC * _NS * _CH)
_NP = 10240
_RPT = _NP // _NS

_mesh = plsc.VectorSubcoreMesh(core_axis_name="c", subcore_axis_name="s")


def _deg_body(srcr, dstr, zn, out_s, out_d, deg_s, deg_d,
              src_v, dst_v, ones_v):
    c = lax.axis_index("c")
    s = lax.axis_index("s")
    for i in range(_CH // 16):
        ones_v[pl.ds(16 * i, 16)] = jnp.ones((16,), jnp.float32)

    @pl.when(s == 0)
    def _():
        pltpu.sync_copy(zn, deg_s)

    @pl.when(s == 1)
    def _():
        pltpu.sync_copy(zn, deg_d)

    pltpu.sync_copy(srcr.at[c, s], src_v)
    pltpu.sync_copy(dstr.at[c, s], dst_v)
    plsc.subcore_barrier()

    def body(j, carry):
        pltpu.sync_copy(ones_v, deg_s.at[src_v.at[j]], add=True)
        pltpu.sync_copy(ones_v, deg_d.at[dst_v.at[j]], add=True)
        return carry

    lax.fori_loop(0, _NJ, body, 0)
    plsc.subcore_barrier()

    @pl.when(s == 0)
    def _():
        pltpu.sync_copy(deg_s, out_s.at[c])

    @pl.when(s == 1)
    def _():
        pltpu.sync_copy(deg_d, out_d.at[c])


_sc_degrees = functools.partial(
    pl.kernel,
    _deg_body,
    out_type=(
        jax.ShapeDtypeStruct((_NC, _N), jnp.float32),
        jax.ShapeDtypeStruct((_NC, _N), jnp.float32),
    ),
    mesh=_mesh,
    scratch_types=[
        pltpu.VMEM_SHARED((_N,), jnp.float32),
        pltpu.VMEM_SHARED((_N,), jnp.float32),
        pltpu.VMEM((_NJ, _CH), jnp.int32),
        pltpu.VMEM((_NJ, _CH), jnp.int32),
        pltpu.VMEM((_CH,), jnp.float32),
    ],
)()


def _agg_body(h, srcr, dstr, zr, out, agg, src_v, dst_v, rows_v, sem):
    c = lax.axis_index("c")
    s = lax.axis_index("s")
    base = s * _RPT
    for k in range(_RPT // 128):
        pltpu.sync_copy(zr, agg.at[pl.ds(base + k * 128, 128)])
    pltpu.sync_copy(srcr.at[c, s], src_v)
    pltpu.sync_copy(dstr.at[c, s], dst_v)
    plsc.subcore_barrier()

    def body(j, carry):
        pltpu.async_copy(h.at[src_v.at[j]], rows_v, sem).wait()
        pltpu.sync_copy(rows_v, agg.at[dst_v.at[j]], add=True)
        return carry

    lax.fori_loop(0, _NJ, body, 0)
    plsc.subcore_barrier()
    pltpu.sync_copy(agg.at[pl.ds(base, _RPT)], out.at[c, pl.ds(base, _RPT)])


_sc_scatter = functools.partial(
    pl.kernel,
    _agg_body,
    out_type=jax.ShapeDtypeStruct((_NC, _NP, _D), jnp.float32),
    mesh=_mesh,
    scratch_types=[
        pltpu.VMEM_SHARED((_NP, _D), jnp.float32),
        pltpu.VMEM((_NJ, _CH), jnp.int32),
        pltpu.VMEM((_NJ, _CH), jnp.int32),
        pltpu.VMEM((_CH, _D), jnp.float32),
        pltpu.SemaphoreType.DMA,
    ],
)()


def _scale_body(x_ref, d0_ref, d1_ref, h_ref):
    d = d0_ref[...] + d1_ref[...]
    norm = jnp.where(d > 0.0, lax.rsqrt(d), 0.0)
    h_ref[...] = x_ref[...] * norm


_tc_scale = pl.pallas_call(
    _scale_body,
    out_shape=jax.ShapeDtypeStruct((_N, _D), jnp.float32),
)


def _head_body(p_ref, d0_ref, d1_ref, wgc_ref, bgc_ref, wf_ref, bf_ref,
               wc_ref, bc_ref, y_ref, z_ref):
    d = d0_ref[...] + d1_ref[...]
    nd = jnp.where(d > 0.0, lax.rsqrt(d), 0.0)
    agg = (p_ref[0, :_N] + p_ref[1, :_N]) * nd
    z0 = jnp.dot(agg, wgc_ref[...], precision=lax.Precision.HIGHEST,
                 preferred_element_type=jnp.float32) + bgc_ref[...]
    z = jnp.maximum(
        jnp.dot(z0, wf_ref[...], precision=lax.Precision.HIGHEST,
                preferred_element_type=jnp.float32) + bf_ref[...], 0.0)
    y = jnp.dot(z, wc_ref[...], precision=lax.Precision.HIGHEST,
                preferred_element_type=jnp.float32) + bc_ref[...]
    y_ref[...] = y
    z_ref[...] = z


_tc_head = pl.pallas_call(
    _head_body,
    out_shape=(
        jax.ShapeDtypeStruct((_N, 1), jnp.float32),
        jax.ShapeDtypeStruct((_N, _H), jnp.float32),
    ),
)


def kernel(x, edge_index, W_gc, b_gc, W_fair, b_fair, W_cls, b_cls):
    src = edge_index[0].reshape(_NC, _NS, _NJ, _CH)
    dst = edge_index[1].reshape(_NC, _NS, _NJ, _CH)
    zn = jnp.zeros((_N,), jnp.float32)
    zr = jnp.zeros((128, _D), jnp.float32)

    deg_s, deg_d = _sc_degrees(src, dst, zn)
    ds0 = deg_s[0].reshape(_N, 1)
    ds1 = deg_s[1].reshape(_N, 1)
    dd0 = deg_d[0].reshape(_N, 1)
    dd1 = deg_d[1].reshape(_N, 1)

    h = _tc_scale(x, ds0, ds1)
    p = _sc_scatter(h, src, dst, zr)
    y, z = _tc_head(p, dd0, dd1, W_gc, b_gc, W_fair, b_fair, W_cls, b_cls)
    return (y, z)


# speedup vs baseline: 8.7399x; 1.1750x over previous
import functools

import jax
import jax.numpy as jnp
from jax import lax
from jax.experimental import pallas as pl
from jax.experimental.pallas import tpu as pltpu
from jax.experimental.pallas import tpu_sc as plsc

_N = 10000
_E = 320000
_D = 128
_H = 128

_NC = 2
_NS = 16
_CH = 80
_NJ = _E // (_NC * _NS * _CH)

_mesh = plsc.VectorSubcoreMesh(core_axis_name="c", subcore_axis_name="s")


def _deg_body(srcr, dstr, zn, out_s, out_d, deg_s, deg_d,
              src_v, dst_v, ones_v):
    c = lax.axis_index("c")
    s = lax.axis_index("s")
    for i in range(_CH // 16):
        ones_v[pl.ds(16 * i, 16)] = jnp.ones((16,), jnp.float32)

    @pl.when(s == 0)
    def _():
        pltpu.sync_copy(zn, deg_s)

    @pl.when(s == 1)
    def _():
        pltpu.sync_copy(zn, deg_d)

    pltpu.sync_copy(srcr.at[c, s], src_v)
    pltpu.sync_copy(dstr.at[c, s], dst_v)
    plsc.subcore_barrier()

    def body(j, carry):
        pltpu.sync_copy(ones_v, deg_s.at[src_v.at[j]], add=True)
        pltpu.sync_copy(ones_v, deg_d.at[dst_v.at[j]], add=True)
        return carry

    lax.fori_loop(0, _NJ, body, 0)
    plsc.subcore_barrier()

    @pl.when(s == 0)
    def _():
        pltpu.sync_copy(deg_s, out_s.at[c])

    @pl.when(s == 1)
    def _():
        pltpu.sync_copy(deg_d, out_d.at[c])


_sc_degrees = functools.partial(
    pl.kernel,
    _deg_body,
    out_type=(
        jax.ShapeDtypeStruct((_NC, _N), jnp.float32),
        jax.ShapeDtypeStruct((_NC, _N), jnp.float32),
    ),
    mesh=_mesh,
    scratch_types=[
        pltpu.VMEM_SHARED((_N,), jnp.float32),
        pltpu.VMEM_SHARED((_N,), jnp.float32),
        pltpu.VMEM((_NJ, _CH), jnp.int32),
        pltpu.VMEM((_NJ, _CH), jnp.int32),
        pltpu.VMEM((_CH,), jnp.float32),
    ],
)()


def _agg_body(h, srcr, dstr1, dstr2, zr, out, agg, src_v, dst_v,
              rows_a, rows_b, sem):
    c = lax.axis_index("c")
    s = lax.axis_index("s")
    base = jnp.where(s < 14, s * 632, 8848 + (s - 14) * 576)
    for k in range(4):
        pltpu.sync_copy(zr, agg.at[pl.ds(base + k * 128, 128)])

    @pl.when(s < 14)
    def _():
        pltpu.sync_copy(zr.at[pl.ds(0, 120)], agg.at[pl.ds(base + 512, 120)])

    @pl.when(s >= 14)
    def _():
        pltpu.sync_copy(zr.at[pl.ds(0, 64)], agg.at[pl.ds(base + 512, 64)])

    pltpu.sync_copy(srcr.at[c, s], src_v)
    pltpu.sync_copy(dstr1.at[c, s], dst_v)
    plsc.subcore_barrier()

    pltpu.async_copy(h.at[src_v.at[0]], rows_a, sem)

    def pair(t, carry):
        ja = 2 * t
        jb = ja + 1

        @pl.when(ja == 64)
        def _():
            pltpu.sync_copy(dstr2.at[c, s], dst_v.at[pl.ds(0, _NJ - 64)])

        ila = jnp.where(ja >= 64, ja - 64, ja)
        ilb = ila + 1
        pltpu.make_async_copy(h.at[src_v.at[ja]], rows_a, sem).wait()
        pltpu.async_copy(h.at[src_v.at[jb]], rows_b, sem)
        pltpu.sync_copy(rows_a, agg.at[dst_v.at[ila]], add=True)
        pltpu.make_async_copy(h.at[src_v.at[jb]], rows_b, sem).wait()
        pltpu.async_copy(h.at[src_v.at[ja + 2]], rows_a, sem)
        pltpu.sync_copy(rows_b, agg.at[dst_v.at[ilb]], add=True)
        return carry

    lax.fori_loop(0, (_NJ - 1) // 2, pair, 0)
    pltpu.make_async_copy(h.at[src_v.at[_NJ - 1]], rows_a, sem).wait()
    pltpu.sync_copy(rows_a, agg.at[dst_v.at[_NJ - 1 - 64]], add=True)
    plsc.subcore_barrier()

    @pl.when(s < 14)
    def _():
        pltpu.sync_copy(agg.at[pl.ds(base, 632)], out.at[c, pl.ds(base, 632)])

    @pl.when(s >= 14)
    def _():
        pltpu.sync_copy(agg.at[pl.ds(base, 576)], out.at[c, pl.ds(base, 576)])


_sc_scatter = functools.partial(
    pl.kernel,
    _agg_body,
    out_type=jax.ShapeDtypeStruct((_NC, _N, _D), jnp.float32),
    mesh=_mesh,
    scratch_types=[
        pltpu.VMEM_SHARED((_N, _D), jnp.float32),
        pltpu.VMEM((_NJ, _CH), jnp.int32),
        pltpu.VMEM((64, _CH), jnp.int32),
        pltpu.VMEM((_CH, _D), jnp.float32),
        pltpu.VMEM((_CH, _D), jnp.float32),
        pltpu.SemaphoreType.DMA,
    ],
)()


def _scale_body(x_ref, d0_ref, d1_ref, h_ref):
    d = d0_ref[...] + d1_ref[...]
    norm = jnp.where(d > 0.0, lax.rsqrt(d), 0.0)
    h_ref[...] = x_ref[...] * norm


_tc_scale = pl.pallas_call(
    _scale_body,
    out_shape=jax.ShapeDtypeStruct((_N, _D), jnp.float32),
)


def _head_body(p_ref, d0_ref, d1_ref, wgc_ref, bgc_ref, wf_ref, bf_ref,
               wc_ref, bc_ref, y_ref, z_ref):
    d = d0_ref[...] + d1_ref[...]
    nd = jnp.where(d > 0.0, lax.rsqrt(d), 0.0)
    agg = (p_ref[0] + p_ref[1]) * nd
    z0 = jnp.dot(agg, wgc_ref[...], precision=lax.Precision.HIGHEST,
                 preferred_element_type=jnp.float32) + bgc_ref[...]
    z = jnp.maximum(
        jnp.dot(z0, wf_ref[...], precision=lax.Precision.HIGHEST,
                preferred_element_type=jnp.float32) + bf_ref[...], 0.0)
    y = jnp.dot(z, wc_ref[...], precision=lax.Precision.HIGHEST,
                preferred_element_type=jnp.float32) + bc_ref[...]
    y_ref[...] = y
    z_ref[...] = z


_tc_head = pl.pallas_call(
    _head_body,
    out_shape=(
        jax.ShapeDtypeStruct((_N, 1), jnp.float32),
        jax.ShapeDtypeStruct((_N, _H), jnp.float32),
    ),
)


def kernel(x, edge_index, W_gc, b_gc, W_fair, b_fair, W_cls, b_cls):
    src = edge_index[0].reshape(_NC, _NS, _NJ, _CH)
    dst = edge_index[1].reshape(_NC, _NS, _NJ, _CH)
    zn = jnp.zeros((_N,), jnp.float32)
    zr = jnp.zeros((128, _D), jnp.float32)

    deg_s, deg_d = _sc_degrees(src, dst, zn)
    ds0 = deg_s[0].reshape(_N, 1)
    ds1 = deg_s[1].reshape(_N, 1)
    dd0 = deg_d[0].reshape(_N, 1)
    dd1 = deg_d[1].reshape(_N, 1)

    h = _tc_scale(x, ds0, ds1)
    p = _sc_scatter(h, src, dst[:, :, :64], dst[:, :, 64:], zr)
    y, z = _tc_head(p, dd0, dd1, W_gc, b_gc, W_fair, b_fair, W_cls, b_cls)
    return (y, z)


# speedup vs baseline: 8.7822x; 1.0048x over previous
import functools

import jax
import jax.numpy as jnp
from jax import lax
from jax.experimental import pallas as pl
from jax.experimental.pallas import tpu as pltpu
from jax.experimental.pallas import tpu_sc as plsc

_N = 10000
_E = 320000
_D = 128
_H = 128

_NC = 2
_NS = 16
_CH = 80
_NJ = _E // (_NC * _NS * _CH)

_mesh = plsc.VectorSubcoreMesh(core_axis_name="c", subcore_axis_name="s")


def _deg_body(srcr, dstr, zn, out_s, out_d, deg_s, deg_d,
              src_v, dst_v, ones_v):
    c = lax.axis_index("c")
    s = lax.axis_index("s")
    for i in range(_CH // 16):
        ones_v[pl.ds(16 * i, 16)] = jnp.ones((16,), jnp.float32)

    @pl.when(s == 0)
    def _():
        pltpu.sync_copy(zn, deg_s)

    @pl.when(s == 1)
    def _():
        pltpu.sync_copy(zn, deg_d)

    pltpu.sync_copy(srcr.at[c, s], src_v)
    pltpu.sync_copy(dstr.at[c, s], dst_v)
    plsc.subcore_barrier()

    def body(j, carry):
        pltpu.sync_copy(ones_v, deg_s.at[src_v.at[j]], add=True)
        pltpu.sync_copy(ones_v, deg_d.at[dst_v.at[j]], add=True)
        return carry

    lax.fori_loop(0, _NJ, body, 0)
    plsc.subcore_barrier()

    @pl.when(s == 0)
    def _():
        pltpu.sync_copy(deg_s, out_s.at[c])

    @pl.when(s == 1)
    def _():
        pltpu.sync_copy(deg_d, out_d.at[c])


_sc_degrees = functools.partial(
    pl.kernel,
    _deg_body,
    out_type=(
        jax.ShapeDtypeStruct((_NC, _N), jnp.float32),
        jax.ShapeDtypeStruct((_NC, _N), jnp.float32),
    ),
    mesh=_mesh,
    scratch_types=[
        pltpu.VMEM_SHARED((_N,), jnp.float32),
        pltpu.VMEM_SHARED((_N,), jnp.float32),
        pltpu.VMEM((_NJ, _CH), jnp.int32),
        pltpu.VMEM((_NJ, _CH), jnp.int32),
        pltpu.VMEM((_CH,), jnp.float32),
    ],
)()


def _agg_body(h, srcr, dstr1, dstr2, zr, out, agg, src_v, dst_v,
              rows_a, rows_b, gsem, ssem):
    c = lax.axis_index("c")
    s = lax.axis_index("s")
    base = jnp.where(s < 14, s * 632, 8848 + (s - 14) * 576)
    for k in range(4):
        pltpu.async_copy(zr, agg.at[pl.ds(base + k * 128, 128)], gsem)

    @pl.when(s < 14)
    def _():
        pltpu.async_copy(zr.at[pl.ds(0, 120)],
                         agg.at[pl.ds(base + 512, 120)], gsem)

    @pl.when(s >= 14)
    def _():
        pltpu.async_copy(zr.at[pl.ds(0, 64)],
                         agg.at[pl.ds(base + 512, 64)], gsem)

    pltpu.async_copy(srcr.at[c, s], src_v, gsem)
    pltpu.async_copy(dstr1.at[c, s], dst_v, gsem)
    for k in range(4):
        pltpu.make_async_copy(zr, agg.at[pl.ds(base + k * 128, 128)],
                              gsem).wait()

    @pl.when(s < 14)
    def _():
        pltpu.make_async_copy(zr.at[pl.ds(0, 120)],
                              agg.at[pl.ds(base + 512, 120)], gsem).wait()

    @pl.when(s >= 14)
    def _():
        pltpu.make_async_copy(zr.at[pl.ds(0, 64)],
                              agg.at[pl.ds(base + 512, 64)], gsem).wait()

    pltpu.make_async_copy(srcr.at[c, s], src_v, gsem).wait()
    pltpu.make_async_copy(dstr1.at[c, s], dst_v, gsem).wait()
    plsc.subcore_barrier()

    def _gather(j, buf):
        pltpu.async_copy(h.at[src_v.at[j]], buf, gsem)

    def _gwait(j, buf):
        pltpu.make_async_copy(h.at[src_v.at[j]], buf, gsem).wait()

    def _scat(il, buf):
        pltpu.async_copy(buf, agg.at[dst_v.at[il]], ssem, add=True)

    def _swait(il, buf):
        pltpu.make_async_copy(buf, agg.at[dst_v.at[il]], ssem).wait()

    _gather(0, rows_a)

    def pair(t, carry):
        ja = 2 * t
        jb = ja + 1

        @pl.when(ja == 64)
        def _():
            pltpu.sync_copy(dstr2.at[c, s], dst_v.at[pl.ds(0, _NJ - 64)])

        ila = jnp.where(ja >= 64, ja - 64, ja)
        ilb = ila + 1
        ilprev = jnp.where(ja == 64, 63, ila - 1)
        _gwait(ja, rows_a)

        @pl.when(t > 0)
        def _():
            _swait(ilprev, rows_b)

        _scat(ila, rows_a)
        _gather(jb, rows_b)
        _gwait(jb, rows_b)
        _swait(ila, rows_a)
        _scat(ilb, rows_b)
        _gather(ja + 2, rows_a)
        return carry

    lax.fori_loop(0, (_NJ - 1) // 2, pair, 0)
    _gwait(_NJ - 1, rows_a)
    _swait(_NJ - 2 - 64, rows_b)
    _scat(_NJ - 1 - 64, rows_a)
    _swait(_NJ - 1 - 64, rows_a)
    plsc.subcore_barrier()

    @pl.when(s < 14)
    def _():
        pltpu.sync_copy(agg.at[pl.ds(base, 632)], out.at[c, pl.ds(base, 632)])

    @pl.when(s >= 14)
    def _():
        pltpu.sync_copy(agg.at[pl.ds(base, 576)], out.at[c, pl.ds(base, 576)])


_sc_scatter = functools.partial(
    pl.kernel,
    _agg_body,
    out_type=jax.ShapeDtypeStruct((_NC, _N, _D), jnp.float32),
    mesh=_mesh,
    scratch_types=[
        pltpu.VMEM_SHARED((_N, _D), jnp.float32),
        pltpu.VMEM((_NJ, _CH), jnp.int32),
        pltpu.VMEM((64, _CH), jnp.int32),
        pltpu.VMEM((_CH, _D), jnp.float32),
        pltpu.VMEM((_CH, _D), jnp.float32),
        pltpu.SemaphoreType.DMA,
        pltpu.SemaphoreType.DMA,
    ],
)()


def _scale_body(x_ref, d0_ref, d1_ref, h_ref):
    d = d0_ref[...] + d1_ref[...]
    norm = jnp.where(d > 0.0, lax.rsqrt(d), 0.0)
    h_ref[...] = x_ref[...] * norm


_tc_scale = pl.pallas_call(
    _scale_body,
    out_shape=jax.ShapeDtypeStruct((_N, _D), jnp.float32),
)


def _head_body(p_ref, d0_ref, d1_ref, wgc_ref, bgc_ref, wf_ref, bf_ref,
               wc_ref, bc_ref, y_ref, z_ref):
    d = d0_ref[...] + d1_ref[...]
    nd = jnp.where(d > 0.0, lax.rsqrt(d), 0.0)
    agg = (p_ref[0] + p_ref[1]) * nd
    z0 = jnp.dot(agg, wgc_ref[...], precision=lax.Precision.HIGHEST,
                 preferred_element_type=jnp.float32) + bgc_ref[...]
    z = jnp.maximum(
        jnp.dot(z0, wf_ref[...], precision=lax.Precision.HIGHEST,
                preferred_element_type=jnp.float32) + bf_ref[...], 0.0)
    y = jnp.dot(z, wc_ref[...], precision=lax.Precision.HIGHEST,
                preferred_element_type=jnp.float32) + bc_ref[...]
    y_ref[...] = y
    z_ref[...] = z


_tc_head = pl.pallas_call(
    _head_body,
    out_shape=(
        jax.ShapeDtypeStruct((_N, 1), jnp.float32),
        jax.ShapeDtypeStruct((_N, _H), jnp.float32),
    ),
)


def kernel(x, edge_index, W_gc, b_gc, W_fair, b_fair, W_cls, b_cls):
    src = edge_index[0].reshape(_NC, _NS, _NJ, _CH)
    dst = edge_index[1].reshape(_NC, _NS, _NJ, _CH)
    zn = jnp.zeros((_N,), jnp.float32)
    zr = jnp.zeros((128, _D), jnp.float32)

    deg_s, deg_d = _sc_degrees(src, dst, zn)
    ds0 = deg_s[0].reshape(_N, 1)
    ds1 = deg_s[1].reshape(_N, 1)
    dd0 = deg_d[0].reshape(_N, 1)
    dd1 = deg_d[1].reshape(_N, 1)

    h = _tc_scale(x, ds0, ds1)
    p = _sc_scatter(h, src, dst[:, :, :64], dst[:, :, 64:], zr)
    y, z = _tc_head(p, dd0, dd1, W_gc, b_gc, W_fair, b_fair, W_cls, b_cls)
    return (y, z)


# speedup vs baseline: 9.3107x; 1.0602x over previous
import functools

import jax
import jax.numpy as jnp
from jax import lax
from jax.experimental import pallas as pl
from jax.experimental.pallas import tpu as pltpu
from jax.experimental.pallas import tpu_sc as plsc

_N = 10000
_E = 320000
_D = 128
_H = 128

_NC = 2
_NS = 16
_CH = 80
_NJ = _E // (_NC * _NS * _CH)

_mesh = plsc.VectorSubcoreMesh(core_axis_name="c", subcore_axis_name="s")


def _deg_body(srcr, dstr, zn, out_s, out_d, deg_s, deg_d,
              src_v, dst_v, ones_v):
    c = lax.axis_index("c")
    s = lax.axis_index("s")
    for i in range(_CH // 16):
        ones_v[pl.ds(16 * i, 16)] = jnp.ones((16,), jnp.float32)

    @pl.when(s == 0)
    def _():
        pltpu.sync_copy(zn, deg_s)

    @pl.when(s == 1)
    def _():
        pltpu.sync_copy(zn, deg_d)

    pltpu.sync_copy(srcr.at[c, s], src_v)
    pltpu.sync_copy(dstr.at[c, s], dst_v)
    plsc.subcore_barrier()

    def body(j, carry):
        pltpu.sync_copy(ones_v, deg_s.at[src_v.at[j]], add=True)
        pltpu.sync_copy(ones_v, deg_d.at[dst_v.at[j]], add=True)
        return carry

    lax.fori_loop(0, _NJ, body, 0)
    plsc.subcore_barrier()

    @pl.when(s == 0)
    def _():
        pltpu.sync_copy(deg_s, out_s.at[c])

    @pl.when(s == 1)
    def _():
        pltpu.sync_copy(deg_d, out_d.at[c])


_sc_degrees = functools.partial(
    pl.kernel,
    _deg_body,
    out_type=(
        jax.ShapeDtypeStruct((_NC, _N), jnp.float32),
        jax.ShapeDtypeStruct((_NC, _N), jnp.float32),
    ),
    mesh=_mesh,
    scratch_types=[
        pltpu.VMEM_SHARED((_N,), jnp.float32),
        pltpu.VMEM_SHARED((_N,), jnp.float32),
        pltpu.VMEM((_NJ, _CH), jnp.int32),
        pltpu.VMEM((_NJ, _CH), jnp.int32),
        pltpu.VMEM((_CH,), jnp.float32),
    ],
)()


def _agg_body(h, srcr, dstr1, dstr2, zr, out, agg, src_v, dst_v,
              rows_a, rows_b, gsem, ssem):
    c = lax.axis_index("c")
    s = lax.axis_index("s")
    base = jnp.where(s < 14, s * 632, 8848 + (s - 14) * 576)
    for k in range(4):
        pltpu.async_copy(zr, agg.at[pl.ds(base + k * 128, 128)], gsem)

    @pl.when(s < 14)
    def _():
        pltpu.async_copy(zr.at[pl.ds(0, 120)],
                         agg.at[pl.ds(base + 512, 120)], gsem)

    @pl.when(s >= 14)
    def _():
        pltpu.async_copy(zr.at[pl.ds(0, 64)],
                         agg.at[pl.ds(base + 512, 64)], gsem)

    pltpu.async_copy(srcr.at[c, s], src_v, gsem)
    pltpu.async_copy(dstr1.at[c, s], dst_v, gsem)
    for k in range(4):
        pltpu.make_async_copy(zr, agg.at[pl.ds(base + k * 128, 128)],
                              gsem).wait()

    @pl.when(s < 14)
    def _():
        pltpu.make_async_copy(zr.at[pl.ds(0, 120)],
                              agg.at[pl.ds(base + 512, 120)], gsem).wait()

    @pl.when(s >= 14)
    def _():
        pltpu.make_async_copy(zr.at[pl.ds(0, 64)],
                              agg.at[pl.ds(base + 512, 64)], gsem).wait()

    pltpu.make_async_copy(srcr.at[c, s], src_v, gsem).wait()
    pltpu.make_async_copy(dstr1.at[c, s], dst_v, gsem).wait()
    plsc.subcore_barrier()

    def _gather(j, buf):
        pltpu.async_copy(h.at[src_v.at[j]], buf, gsem)

    def _gwait(j, buf):
        pltpu.make_async_copy(h.at[src_v.at[j]], buf, gsem).wait()

    def _scat(il, buf):
        pltpu.async_copy(buf, agg.at[dst_v.at[il]], ssem, add=True)

    def _swait(il, buf):
        pltpu.make_async_copy(buf, agg.at[dst_v.at[il]], ssem).wait()

    _gather(0, rows_a)

    def pair(t, carry):
        ja = 2 * t
        jb = ja + 1

        @pl.when(ja == 64)
        def _():
            pltpu.sync_copy(dstr2.at[c, s], dst_v.at[pl.ds(0, _NJ - 64)])

        ila = jnp.where(ja >= 64, ja - 64, ja)
        ilb = ila + 1
        ilprev = jnp.where(ja == 64, 63, ila - 1)
        _gwait(ja, rows_a)

        @pl.when(t > 0)
        def _():
            _swait(ilprev, rows_b)

        _scat(ila, rows_a)
        _gather(jb, rows_b)
        _gwait(jb, rows_b)
        _swait(ila, rows_a)
        _scat(ilb, rows_b)
        _gather(ja + 2, rows_a)
        return carry

    lax.fori_loop(0, (_NJ - 1) // 2, pair, 0)
    _gwait(_NJ - 1, rows_a)
    _swait(_NJ - 2 - 64, rows_b)
    _scat(_NJ - 1 - 64, rows_a)
    _swait(_NJ - 1 - 64, rows_a)
    plsc.subcore_barrier()

    @pl.when(s < 14)
    def _():
        pltpu.sync_copy(agg.at[pl.ds(base, 632)], out.at[c, pl.ds(base, 632)])

    @pl.when(s >= 14)
    def _():
        pltpu.sync_copy(agg.at[pl.ds(base, 576)], out.at[c, pl.ds(base, 576)])


_sc_scatter = functools.partial(
    pl.kernel,
    _agg_body,
    out_type=jax.ShapeDtypeStruct((_NC, _N, _D), jnp.float32),
    mesh=_mesh,
    scratch_types=[
        pltpu.VMEM_SHARED((_N, _D), jnp.float32),
        pltpu.VMEM((_NJ, _CH), jnp.int32),
        pltpu.VMEM((64, _CH), jnp.int32),
        pltpu.VMEM((_CH, _D), jnp.float32),
        pltpu.VMEM((_CH, _D), jnp.float32),
        pltpu.SemaphoreType.DMA,
        pltpu.SemaphoreType.DMA,
    ],
)()


_BLK = 2000


def _scale_body(x_ref, d0_ref, d1_ref, h_ref):
    d = d0_ref[...] + d1_ref[...]
    norm = jnp.where(d > 0.0, lax.rsqrt(d), 0.0)
    h_ref[...] = x_ref[...] * norm


_tc_scale = pl.pallas_call(
    _scale_body,
    grid=(_N // _BLK,),
    in_specs=[
        pl.BlockSpec((_BLK, _D), lambda i: (i, 0)),
        pl.BlockSpec((_BLK, 1), lambda i: (i, 0)),
        pl.BlockSpec((_BLK, 1), lambda i: (i, 0)),
    ],
    out_specs=pl.BlockSpec((_BLK, _D), lambda i: (i, 0)),
    out_shape=jax.ShapeDtypeStruct((_N, _D), jnp.float32),
)


def _head_body(p_ref, d0_ref, d1_ref, wgc_ref, bgc_ref, wf_ref, bf_ref,
               wc_ref, bc_ref, y_ref, z_ref):
    d = d0_ref[...] + d1_ref[...]
    nd = jnp.where(d > 0.0, lax.rsqrt(d), 0.0)
    agg = (p_ref[0] + p_ref[1]) * nd
    z0 = jnp.dot(agg, wgc_ref[...],
                 preferred_element_type=jnp.float32) + bgc_ref[...]
    z = jnp.maximum(
        jnp.dot(z0, wf_ref[...],
                preferred_element_type=jnp.float32) + bf_ref[...], 0.0)
    y = jnp.dot(z, wc_ref[...],
                preferred_element_type=jnp.float32) + bc_ref[...]
    y_ref[...] = y
    z_ref[...] = z


_tc_head = pl.pallas_call(
    _head_body,
    grid=(_N // _BLK,),
    in_specs=[
        pl.BlockSpec((_NC, _BLK, _D), lambda i: (0, i, 0)),
        pl.BlockSpec((_BLK, 1), lambda i: (i, 0)),
        pl.BlockSpec((_BLK, 1), lambda i: (i, 0)),
        pl.BlockSpec((_D, _H), lambda i: (0, 0)),
        pl.BlockSpec((_H,), lambda i: (0,)),
        pl.BlockSpec((_H, _H), lambda i: (0, 0)),
        pl.BlockSpec((_H,), lambda i: (0,)),
        pl.BlockSpec((_H, 1), lambda i: (0, 0)),
        pl.BlockSpec((1,), lambda i: (0,)),
    ],
    out_specs=(
        pl.BlockSpec((_BLK, 1), lambda i: (i, 0)),
        pl.BlockSpec((_BLK, _H), lambda i: (i, 0)),
    ),
    out_shape=(
        jax.ShapeDtypeStruct((_N, 1), jnp.float32),
        jax.ShapeDtypeStruct((_N, _H), jnp.float32),
    ),
)


def kernel(x, edge_index, W_gc, b_gc, W_fair, b_fair, W_cls, b_cls):
    src = edge_index[0].reshape(_NC, _NS, _NJ, _CH)
    dst = edge_index[1].reshape(_NC, _NS, _NJ, _CH)
    zn = jnp.zeros((_N,), jnp.float32)
    zr = jnp.zeros((128, _D), jnp.float32)

    deg_s, deg_d = _sc_degrees(src, dst, zn)
    ds0 = deg_s[0].reshape(_N, 1)
    ds1 = deg_s[1].reshape(_N, 1)
    dd0 = deg_d[0].reshape(_N, 1)
    dd1 = deg_d[1].reshape(_N, 1)

    h = _tc_scale(x, ds0, ds1)
    p = _sc_scatter(h, src, dst[:, :, :64], dst[:, :, 64:], zr)
    y, z = _tc_head(p, dd0, dd1, W_gc, b_gc, W_fair, b_fair, W_cls, b_cls)
    return (y, z)


# speedup vs baseline: 9.7938x; 1.0519x over previous
import functools

import jax
import jax.numpy as jnp
from jax import lax
from jax.experimental import pallas as pl
from jax.experimental.pallas import tpu as pltpu
from jax.experimental.pallas import tpu_sc as plsc

_N = 10000
_E = 320000
_D = 128
_H = 128

_NC = 2
_NS = 16
_CH = 80
_NJ = _E // (_NC * _NS * _CH)

_mesh = plsc.VectorSubcoreMesh(core_axis_name="c", subcore_axis_name="s")


def _deg_body(srcr, dstr, zn, out_s, out_d, deg_s, deg_d,
              src_v, dst_v, ones_v, sem):
    c = lax.axis_index("c")
    s = lax.axis_index("s")
    for i in range(_CH // 16):
        ones_v[pl.ds(16 * i, 16)] = jnp.ones((16,), jnp.float32)

    @pl.when(s == 0)
    def _():
        pltpu.sync_copy(zn, deg_s)

    @pl.when(s == 1)
    def _():
        pltpu.sync_copy(zn, deg_d)

    pltpu.sync_copy(srcr.at[c, s], src_v)
    pltpu.sync_copy(dstr.at[c, s], dst_v)
    plsc.subcore_barrier()

    def batch(t, carry):
        for k in range(4):
            j = 4 * t + k
            pltpu.async_copy(ones_v, deg_s.at[src_v.at[j]], sem, add=True)
            pltpu.async_copy(ones_v, deg_d.at[dst_v.at[j]], sem, add=True)
        for k in range(4):
            j = 4 * t + k
            pltpu.make_async_copy(ones_v, deg_s.at[src_v.at[j]], sem).wait()
            pltpu.make_async_copy(ones_v, deg_d.at[dst_v.at[j]], sem).wait()
        return carry

    lax.fori_loop(0, _NJ // 4, batch, 0)
    pltpu.sync_copy(ones_v, deg_s.at[src_v.at[_NJ - 1]], add=True)
    pltpu.sync_copy(ones_v, deg_d.at[dst_v.at[_NJ - 1]], add=True)
    plsc.subcore_barrier()

    @pl.when(s == 0)
    def _():
        pltpu.sync_copy(deg_s, out_s.at[c])

    @pl.when(s == 1)
    def _():
        pltpu.sync_copy(deg_d, out_d.at[c])


_sc_degrees = functools.partial(
    pl.kernel,
    _deg_body,
    out_type=(
        jax.ShapeDtypeStruct((_NC, _N), jnp.float32),
        jax.ShapeDtypeStruct((_NC, _N), jnp.float32),
    ),
    mesh=_mesh,
    scratch_types=[
        pltpu.VMEM_SHARED((_N,), jnp.float32),
        pltpu.VMEM_SHARED((_N,), jnp.float32),
        pltpu.VMEM((_NJ, _CH), jnp.int32),
        pltpu.VMEM((_NJ, _CH), jnp.int32),
        pltpu.VMEM((_CH,), jnp.float32),
        pltpu.SemaphoreType.DMA,
    ],
)()


def _agg_body(h, srcr, dstr1, dstr2, zr, out, agg, src_v, dst_v,
              rows_a, rows_b, gsem, ssem):
    c = lax.axis_index("c")
    s = lax.axis_index("s")
    base = jnp.where(s < 14, s * 632, 8848 + (s - 14) * 576)
    for k in range(4):
        pltpu.async_copy(zr, agg.at[pl.ds(base + k * 128, 128)], gsem)

    @pl.when(s < 14)
    def _():
        pltpu.async_copy(zr.at[pl.ds(0, 120)],
                         agg.at[pl.ds(base + 512, 120)], gsem)

    @pl.when(s >= 14)
    def _():
        pltpu.async_copy(zr.at[pl.ds(0, 64)],
                         agg.at[pl.ds(base + 512, 64)], gsem)

    pltpu.async_copy(srcr.at[c, s], src_v, gsem)
    pltpu.async_copy(dstr1.at[c, s], dst_v, gsem)
    for k in range(4):
        pltpu.make_async_copy(zr, agg.at[pl.ds(base + k * 128, 128)],
                              gsem).wait()

    @pl.when(s < 14)
    def _():
        pltpu.make_async_copy(zr.at[pl.ds(0, 120)],
                              agg.at[pl.ds(base + 512, 120)], gsem).wait()

    @pl.when(s >= 14)
    def _():
        pltpu.make_async_copy(zr.at[pl.ds(0, 64)],
                              agg.at[pl.ds(base + 512, 64)], gsem).wait()

    pltpu.make_async_copy(srcr.at[c, s], src_v, gsem).wait()
    pltpu.make_async_copy(dstr1.at[c, s], dst_v, gsem).wait()
    plsc.subcore_barrier()

    def _gather(j, buf):
        pltpu.async_copy(h.at[src_v.at[j]], buf, gsem)

    def _gwait(j, buf):
        pltpu.make_async_copy(h.at[src_v.at[j]], buf, gsem).wait()

    def _scat(il, buf):
        pltpu.async_copy(buf, agg.at[dst_v.at[il]], ssem, add=True)

    def _swait(il, buf):
        pltpu.make_async_copy(buf, agg.at[dst_v.at[il]], ssem).wait()

    _gather(0, rows_a)

    def pair(t, carry):
        ja = 2 * t
        jb = ja + 1

        @pl.when(ja == 64)
        def _():
            pltpu.sync_copy(dstr2.at[c, s], dst_v.at[pl.ds(0, _NJ - 64)])

        ila = jnp.where(ja >= 64, ja - 64, ja)
        ilb = ila + 1
        ilprev = jnp.where(ja == 64, 63, ila - 1)
        _gwait(ja, rows_a)

        @pl.when(t > 0)
        def _():
            _swait(ilprev, rows_b)

        _scat(ila, rows_a)
        _gather(jb, rows_b)
        _gwait(jb, rows_b)
        _swait(ila, rows_a)
        _scat(ilb, rows_b)
        _gather(ja + 2, rows_a)
        return carry

    lax.fori_loop(0, (_NJ - 1) // 2, pair, 0)
    _gwait(_NJ - 1, rows_a)
    _swait(_NJ - 2 - 64, rows_b)
    _scat(_NJ - 1 - 64, rows_a)
    _swait(_NJ - 1 - 64, rows_a)
    plsc.subcore_barrier()

    @pl.when(s < 14)
    def _():
        pltpu.sync_copy(agg.at[pl.ds(base, 632)], out.at[c, pl.ds(base, 632)])

    @pl.when(s >= 14)
    def _():
        pltpu.sync_copy(agg.at[pl.ds(base, 576)], out.at[c, pl.ds(base, 576)])


_sc_scatter = functools.partial(
    pl.kernel,
    _agg_body,
    out_type=jax.ShapeDtypeStruct((_NC, _N, _D), jnp.float32),
    mesh=_mesh,
    scratch_types=[
        pltpu.VMEM_SHARED((_N, _D), jnp.float32),
        pltpu.VMEM((_NJ, _CH), jnp.int32),
        pltpu.VMEM((64, _CH), jnp.int32),
        pltpu.VMEM((_CH, _D), jnp.float32),
        pltpu.VMEM((_CH, _D), jnp.float32),
        pltpu.SemaphoreType.DMA,
        pltpu.SemaphoreType.DMA,
    ],
)()


_BLK = 2000


def _scale_body(x_ref, d0_ref, d1_ref, h_ref):
    d = d0_ref[...] + d1_ref[...]
    norm = jnp.where(d > 0.0, lax.rsqrt(d), 0.0)
    h_ref[...] = x_ref[...] * norm


_tc_scale = pl.pallas_call(
    _scale_body,
    grid=(_N // _BLK,),
    in_specs=[
        pl.BlockSpec((_BLK, _D), lambda i: (i, 0)),
        pl.BlockSpec((_BLK, 1), lambda i: (i, 0)),
        pl.BlockSpec((_BLK, 1), lambda i: (i, 0)),
    ],
    out_specs=pl.BlockSpec((_BLK, _D), lambda i: (i, 0)),
    out_shape=jax.ShapeDtypeStruct((_N, _D), jnp.float32),
)


def _head_body(p_ref, d0_ref, d1_ref, wgc_ref, bgc_ref, wf_ref, bf_ref,
               wc_ref, bc_ref, y_ref, z_ref):
    d = d0_ref[...] + d1_ref[...]
    nd = jnp.where(d > 0.0, lax.rsqrt(d), 0.0)
    agg = (p_ref[0] + p_ref[1]) * nd
    z0 = jnp.dot(agg, wgc_ref[...],
                 preferred_element_type=jnp.float32) + bgc_ref[...]
    z = jnp.maximum(
        jnp.dot(z0, wf_ref[...],
                preferred_element_type=jnp.float32) + bf_ref[...], 0.0)
    y = jnp.dot(z, wc_ref[...],
                preferred_element_type=jnp.float32) + bc_ref[...]
    y_ref[...] = y
    z_ref[...] = z


_tc_head = pl.pallas_call(
    _head_body,
    grid=(_N // _BLK,),
    in_specs=[
        pl.BlockSpec((_NC, _BLK, _D), lambda i: (0, i, 0)),
        pl.BlockSpec((_BLK, 1), lambda i: (i, 0)),
        pl.BlockSpec((_BLK, 1), lambda i: (i, 0)),
        pl.BlockSpec((_D, _H), lambda i: (0, 0)),
        pl.BlockSpec((_H,), lambda i: (0,)),
        pl.BlockSpec((_H, _H), lambda i: (0, 0)),
        pl.BlockSpec((_H,), lambda i: (0,)),
        pl.BlockSpec((_H, 1), lambda i: (0, 0)),
        pl.BlockSpec((1,), lambda i: (0,)),
    ],
    out_specs=(
        pl.BlockSpec((_BLK, 1), lambda i: (i, 0)),
        pl.BlockSpec((_BLK, _H), lambda i: (i, 0)),
    ),
    out_shape=(
        jax.ShapeDtypeStruct((_N, 1), jnp.float32),
        jax.ShapeDtypeStruct((_N, _H), jnp.float32),
    ),
)


def kernel(x, edge_index, W_gc, b_gc, W_fair, b_fair, W_cls, b_cls):
    src = edge_index[0].reshape(_NC, _NS, _NJ, _CH)
    dst = edge_index[1].reshape(_NC, _NS, _NJ, _CH)
    zn = jnp.zeros((_N,), jnp.float32)
    zr = jnp.zeros((128, _D), jnp.float32)

    deg_s, deg_d = _sc_degrees(src, dst, zn)
    ds0 = deg_s[0].reshape(_N, 1)
    ds1 = deg_s[1].reshape(_N, 1)
    dd0 = deg_d[0].reshape(_N, 1)
    dd1 = deg_d[1].reshape(_N, 1)

    h = _tc_scale(x, ds0, ds1)
    p = _sc_scatter(h, src, dst[:, :, :64], dst[:, :, 64:], zr)
    y, z = _tc_head(p, dd0, dd1, W_gc, b_gc, W_fair, b_fair, W_cls, b_cls)
    return (y, z)


# speedup vs baseline: 9.8218x; 1.0029x over previous
import functools

import jax
import jax.numpy as jnp
from jax import lax
from jax.experimental import pallas as pl
from jax.experimental.pallas import tpu as pltpu
from jax.experimental.pallas import tpu_sc as plsc

_N = 10000
_E = 320000
_D = 128
_H = 128

_NC = 2
_NS = 16
_CH = 80
_NJ = _E // (_NC * _NS * _CH)

_mesh = plsc.VectorSubcoreMesh(core_axis_name="c", subcore_axis_name="s")


def _deg_body(srcr, dstr, zn, out_s, out_d, deg_s, deg_d,
              src_v, dst_v, ones_v, sem):
    c = lax.axis_index("c")
    s = lax.axis_index("s")
    for i in range(_CH // 16):
        ones_v[pl.ds(16 * i, 16)] = jnp.ones((16,), jnp.float32)

    @pl.when(s == 0)
    def _():
        pltpu.sync_copy(zn, deg_s)

    @pl.when(s == 1)
    def _():
        pltpu.sync_copy(zn, deg_d)

    pltpu.sync_copy(srcr.at[c, s], src_v)
    pltpu.sync_copy(dstr.at[c, s], dst_v)
    plsc.subcore_barrier()

    def batch(t, carry):
        for k in range(4):
            j = 4 * t + k
            pltpu.async_copy(ones_v, deg_s.at[src_v.at[j]], sem, add=True)
            pltpu.async_copy(ones_v, deg_d.at[dst_v.at[j]], sem, add=True)
        for k in range(4):
            j = 4 * t + k
            pltpu.make_async_copy(ones_v, deg_s.at[src_v.at[j]], sem).wait()
            pltpu.make_async_copy(ones_v, deg_d.at[dst_v.at[j]], sem).wait()
        return carry

    lax.fori_loop(0, _NJ // 4, batch, 0)
    pltpu.sync_copy(ones_v, deg_s.at[src_v.at[_NJ - 1]], add=True)
    pltpu.sync_copy(ones_v, deg_d.at[dst_v.at[_NJ - 1]], add=True)
    plsc.subcore_barrier()

    @pl.when(s == 0)
    def _():
        pltpu.sync_copy(deg_s, out_s.at[c])

    @pl.when(s == 1)
    def _():
        pltpu.sync_copy(deg_d, out_d.at[c])


_sc_degrees = functools.partial(
    pl.kernel,
    _deg_body,
    out_type=(
        jax.ShapeDtypeStruct((_NC, _N), jnp.float32),
        jax.ShapeDtypeStruct((_NC, _N), jnp.float32),
    ),
    mesh=_mesh,
    scratch_types=[
        pltpu.VMEM_SHARED((_N,), jnp.float32),
        pltpu.VMEM_SHARED((_N,), jnp.float32),
        pltpu.VMEM((_NJ, _CH), jnp.int32),
        pltpu.VMEM((_NJ, _CH), jnp.int32),
        pltpu.VMEM((_CH,), jnp.float32),
        pltpu.SemaphoreType.DMA,
    ],
)()


def _agg_body(h, srcr, dstr, zr, out, agg, src_v, dst_v,
              rows_a, rows_b, gsem, ssem):
    c = lax.axis_index("c")
    s = lax.axis_index("s")
    base = jnp.where(s < 14, s * 632, 8848 + (s - 14) * 576)
    for k in range(4):
        pltpu.async_copy(zr, agg.at[pl.ds(base + k * 128, 128)], gsem)

    @pl.when(s < 14)
    def _():
        pltpu.async_copy(zr.at[pl.ds(0, 120)],
                         agg.at[pl.ds(base + 512, 120)], gsem)

    @pl.when(s >= 14)
    def _():
        pltpu.async_copy(zr.at[pl.ds(0, 64)],
                         agg.at[pl.ds(base + 512, 64)], gsem)

    pltpu.async_copy(srcr.at[c, s], src_v, gsem)
    pltpu.async_copy(dstr.at[c, s, pl.ds(0, 64)], dst_v, gsem)
    for k in range(4):
        pltpu.make_async_copy(zr, agg.at[pl.ds(base + k * 128, 128)],
                              gsem).wait()

    @pl.when(s < 14)
    def _():
        pltpu.make_async_copy(zr.at[pl.ds(0, 120)],
                              agg.at[pl.ds(base + 512, 120)], gsem).wait()

    @pl.when(s >= 14)
    def _():
        pltpu.make_async_copy(zr.at[pl.ds(0, 64)],
                              agg.at[pl.ds(base + 512, 64)], gsem).wait()

    pltpu.make_async_copy(srcr.at[c, s], src_v, gsem).wait()
    pltpu.make_async_copy(dstr.at[c, s, pl.ds(0, 64)], dst_v, gsem).wait()
    plsc.subcore_barrier()

    def _gather(j, buf):
        pltpu.async_copy(h.at[src_v.at[j]], buf, gsem)

    def _gwait(j, buf):
        pltpu.make_async_copy(h.at[src_v.at[j]], buf, gsem).wait()

    def _scat(il, buf):
        pltpu.async_copy(buf, agg.at[dst_v.at[il]], ssem, add=True)

    def _swait(il, buf):
        pltpu.make_async_copy(buf, agg.at[dst_v.at[il]], ssem).wait()

    _gather(0, rows_a)

    def pair(t, carry):
        ja = 2 * t
        jb = ja + 1

        @pl.when(ja == 64)
        def _():
            pltpu.sync_copy(dstr.at[c, s, pl.ds(64, _NJ - 64)],
                            dst_v.at[pl.ds(0, _NJ - 64)])

        ila = jnp.where(ja >= 64, ja - 64, ja)
        ilb = ila + 1
        ilprev = jnp.where(ja == 64, 63, ila - 1)
        _gwait(ja, rows_a)

        @pl.when(t > 0)
        def _():
            _swait(ilprev, rows_b)

        _scat(ila, rows_a)
        _gather(jb, rows_b)
        _gwait(jb, rows_b)
        _swait(ila, rows_a)
        _scat(ilb, rows_b)
        _gather(ja + 2, rows_a)
        return carry

    lax.fori_loop(0, (_NJ - 1) // 2, pair, 0)
    _gwait(_NJ - 1, rows_a)
    _swait(_NJ - 2 - 64, rows_b)
    _scat(_NJ - 1 - 64, rows_a)
    _swait(_NJ - 1 - 64, rows_a)
    plsc.subcore_barrier()

    @pl.when(s < 14)
    def _():
        pltpu.sync_copy(agg.at[pl.ds(base, 632)], out.at[c, pl.ds(base, 632)])

    @pl.when(s >= 14)
    def _():
        pltpu.sync_copy(agg.at[pl.ds(base, 576)], out.at[c, pl.ds(base, 576)])


_sc_scatter = functools.partial(
    pl.kernel,
    _agg_body,
    out_type=jax.ShapeDtypeStruct((_NC, _N, _D), jnp.float32),
    mesh=_mesh,
    scratch_types=[
        pltpu.VMEM_SHARED((_N, _D), jnp.float32),
        pltpu.VMEM((_NJ, _CH), jnp.int32),
        pltpu.VMEM((64, _CH), jnp.int32),
        pltpu.VMEM((_CH, _D), jnp.float32),
        pltpu.VMEM((_CH, _D), jnp.float32),
        pltpu.SemaphoreType.DMA,
        pltpu.SemaphoreType.DMA,
    ],
)()


_BLK = 2000


def _scale_body(x_ref, d0_ref, d1_ref, h_ref):
    d = d0_ref[...] + d1_ref[...]
    norm = jnp.where(d > 0.0, lax.rsqrt(d), 0.0)
    h_ref[...] = x_ref[...] * norm


_tc_scale = pl.pallas_call(
    _scale_body,
    grid=(_N // _BLK,),
    in_specs=[
        pl.BlockSpec((_BLK, _D), lambda i: (i, 0)),
        pl.BlockSpec((_BLK, 1), lambda i: (i, 0)),
        pl.BlockSpec((_BLK, 1), lambda i: (i, 0)),
    ],
    out_specs=pl.BlockSpec((_BLK, _D), lambda i: (i, 0)),
    out_shape=jax.ShapeDtypeStruct((_N, _D), jnp.float32),
)


def _head_body(p_ref, d0_ref, d1_ref, wgc_ref, bgc_ref, wf_ref, bf_ref,
               wc_ref, bc_ref, y_ref, z_ref):
    d = d0_ref[...] + d1_ref[...]
    nd = jnp.where(d > 0.0, lax.rsqrt(d), 0.0)
    agg = (p_ref[0] + p_ref[1]) * nd
    z0 = jnp.dot(agg, wgc_ref[...],
                 preferred_element_type=jnp.float32) + bgc_ref[...]
    z = jnp.maximum(
        jnp.dot(z0, wf_ref[...],
                preferred_element_type=jnp.float32) + bf_ref[...], 0.0)
    y = jnp.dot(z, wc_ref[...],
                preferred_element_type=jnp.float32) + bc_ref[...]
    y_ref[...] = y
    z_ref[...] = z


_tc_head = pl.pallas_call(
    _head_body,
    grid=(_N // _BLK,),
    in_specs=[
        pl.BlockSpec((_NC, _BLK, _D), lambda i: (0, i, 0)),
        pl.BlockSpec((_BLK, 1), lambda i: (i, 0)),
        pl.BlockSpec((_BLK, 1), lambda i: (i, 0)),
        pl.BlockSpec((_D, _H), lambda i: (0, 0)),
        pl.BlockSpec((_H,), lambda i: (0,)),
        pl.BlockSpec((_H, _H), lambda i: (0, 0)),
        pl.BlockSpec((_H,), lambda i: (0,)),
        pl.BlockSpec((_H, 1), lambda i: (0, 0)),
        pl.BlockSpec((1,), lambda i: (0,)),
    ],
    out_specs=(
        pl.BlockSpec((_BLK, 1), lambda i: (i, 0)),
        pl.BlockSpec((_BLK, _H), lambda i: (i, 0)),
    ),
    out_shape=(
        jax.ShapeDtypeStruct((_N, 1), jnp.float32),
        jax.ShapeDtypeStruct((_N, _H), jnp.float32),
    ),
)


def kernel(x, edge_index, W_gc, b_gc, W_fair, b_fair, W_cls, b_cls):
    src = edge_index[0].reshape(_NC, _NS, _NJ, _CH)
    dst = edge_index[1].reshape(_NC, _NS, _NJ, _CH)
    zn = jnp.zeros((_N,), jnp.float32)
    zr = jnp.zeros((128, _D), jnp.float32)

    deg_s, deg_d = _sc_degrees(src, dst, zn)
    ds0 = deg_s[0].reshape(_N, 1)
    ds1 = deg_s[1].reshape(_N, 1)
    dd0 = deg_d[0].reshape(_N, 1)
    dd1 = deg_d[1].reshape(_N, 1)

    h = _tc_scale(x, ds0, ds1)
    p = _sc_scatter(h, src, dst, zr)
    y, z = _tc_head(p, dd0, dd1, W_gc, b_gc, W_fair, b_fair, W_cls, b_cls)
    return (y, z)
